# baseline (speedup 1.0000x reference)
import functools

import jax
import jax.numpy as jnp
from jax import lax
from jax.experimental import pallas as pl
from jax.experimental.pallas import tpu as pltpu
from jax.experimental.pallas import tpu_sc as plsc

B = 4
N = 10000
NT = B * N
D = 128
E = 160000

NC = 2
NS = 16
EPT = E // NS
W = 80
NWIN = EPT // W
NWT = 10
RPT = N // NWT
NBUF = 3


def _segsum_body(half, h_hbm, e_hbm, z_hbm, agg_hbm, src_f, dst_f, rows0,
                 rows1, rows2, acc_sh, g0, g1, g2, t0, t1, t2):
    c = lax.axis_index("c")
    s = lax.axis_index("s")
    rows = (rows0, rows1, rows2)
    gsem = (g0, g1, g2)
    ssem = (t0, t1, t2)
    if True:
        b = 2 * half + c

        off = b * 2 * E + s * EPT
        pltpu.async_copy(e_hbm.at[pl.ds(off, EPT)], src_f, g0)
        pltpu.async_copy(e_hbm.at[pl.ds(off + E, EPT)], dst_f, g1)

        @pl.when(s < NWT)
        def _zero_slice():
            pltpu.async_copy(z_hbm.at[pl.ds(s * RPT, RPT)],
                             acc_sh.at[pl.ds(s * RPT, RPT)], g2)

        pltpu.make_async_copy(e_hbm.at[pl.ds(off, EPT)], src_f, g0).wait()
        boff = c * N

        @pl.loop(0, EPT // 16)
        def _off(i):
            src_f[pl.ds(i * 16, 16)] = src_f[pl.ds(i * 16, 16)] + boff

        pltpu.make_async_copy(e_hbm.at[pl.ds(off + E, EPT)], dst_f, g1).wait()

        @pl.when(s < NWT)
        def _zero_wait():
            pltpu.make_async_copy(z_hbm.at[pl.ds(s * RPT, RPT)],
                                  acc_sh.at[pl.ds(s * RPT, RPT)], g2).wait()

        plsc.subcore_barrier()

        def _src(w):
            return src_f.at[pl.ds(w * W, W)]

        def _dst(w):
            return dst_f.at[pl.ds(w * W, W)]

        for j in range(NBUF):
            pltpu.async_copy(h_hbm.at[_src(j)], rows[j], gsem[j])

        @pl.loop(0, NWIN // NBUF - 1)
        def _ring(k):
            w0 = NBUF * k
            for j in range(NBUF):
                pltpu.make_async_copy(
                    h_hbm.at[_src(w0 + j)], rows[j], gsem[j]).wait()
                pltpu.async_copy(
                    rows[j], acc_sh.at[_dst(w0 + j)], ssem[j], add=True)
            for j in range(NBUF):
                pltpu.make_async_copy(
                    rows[j], acc_sh.at[_dst(w0 + j)], ssem[j]).wait()
                pltpu.async_copy(
                    h_hbm.at[_src(w0 + j + NBUF)], rows[j], gsem[j])

        wlast = (NWIN // NBUF - 1) * NBUF
        for j in range(NBUF):
            pltpu.make_async_copy(
                h_hbm.at[_src(wlast + j)], rows[j], gsem[j]).wait()
            pltpu.async_copy(
                rows[j], acc_sh.at[_dst(wlast + j)], ssem[j], add=True)
        for j in range(NBUF):
            pltpu.make_async_copy(
                rows[j], acc_sh.at[_dst(wlast + j)], ssem[j]).wait()
        for j in range(NWIN % NBUF):
            w = wlast + NBUF + j
            pltpu.async_copy(h_hbm.at[_src(w)], rows[j], gsem[j]).wait()
            pltpu.sync_copy(rows[j], acc_sh.at[_dst(w)], add=True)

        plsc.subcore_barrier()

        @pl.when(s < NWT)
        def _writeback():
            pltpu.sync_copy(acc_sh.at[pl.ds(s * RPT, RPT)],
                            agg_hbm.at[pl.ds(c * N + s * RPT, RPT)])


@functools.cache
def _make_segsum(half):
    return pl.kernel(
        functools.partial(_segsum_body, half),
        out_type=jax.ShapeDtypeStruct((2 * N, D), jnp.float32),
        mesh=plsc.VectorSubcoreMesh(
            core_axis_name="c", subcore_axis_name="s",
            num_cores=NC, num_subcores=NS,
        ),
        scratch_types=[
            pltpu.VMEM((EPT,), jnp.int32),
            pltpu.VMEM((EPT,), jnp.int32),
            pltpu.VMEM((W, D), jnp.float32),
            pltpu.VMEM((W, D), jnp.float32),
            pltpu.VMEM((W, D), jnp.float32),
            pltpu.VMEM_SHARED((N, D), jnp.float32),
            pltpu.SemaphoreType.DMA,
            pltpu.SemaphoreType.DMA,
            pltpu.SemaphoreType.DMA,
            pltpu.SemaphoreType.DMA,
            pltpu.SemaphoreType.DMA,
            pltpu.SemaphoreType.DMA,
        ],
    )


def _segsum(half, h_half, ef):
    return _make_segsum(half)(h_half, ef, jnp.zeros((N, D), jnp.float32))


ROWS_BLK = 2000


def _gin_body(agg_ref, w_ref, b_ref, g_ref, be_ref, out_ref):
    z = jnp.dot(agg_ref[...], w_ref[...], preferred_element_type=jnp.float32)
    z = z + b_ref[...]
    mu = jnp.mean(z, axis=1, keepdims=True)
    var = jnp.mean((z - mu) ** 2, axis=1, keepdims=True)
    zn = (z - mu) * lax.rsqrt(var + 1e-5) * g_ref[...] + be_ref[...]
    out_ref[...] = jnp.maximum(zn, 0.0)


def _gin_dense(agg, w, b, g, be):
    nrows = agg.shape[0]
    return pl.pallas_call(
        _gin_body,
        grid=(nrows // ROWS_BLK,),
        in_specs=[
            pl.BlockSpec((ROWS_BLK, D), lambda i: (i, 0)),
            pl.BlockSpec((D, D), lambda i: (0, 0)),
            pl.BlockSpec((1, D), lambda i: (0, 0)),
            pl.BlockSpec((1, D), lambda i: (0, 0)),
            pl.BlockSpec((1, D), lambda i: (0, 0)),
        ],
        out_specs=pl.BlockSpec((ROWS_BLK, D), lambda i: (i, 0)),
        out_shape=jax.ShapeDtypeStruct((nrows, D), jnp.float32),
    )(agg, w, b.reshape(1, D), g.reshape(1, D), be.reshape(1, D))


def _final_body(agg_ref, x_ref, h1_ref, h2_ref, cw_ref, cb_ref, cg_ref,
                cbe_ref, w0_ref, w1_ref, w2_ref, w3_ref, bc1_ref, sg_ref,
                bnb_ref, wc2_ref, out_ref):
    i = pl.program_id(0)
    z = jnp.dot(agg_ref[...], cw_ref[...], preferred_element_type=jnp.float32)
    z = z + cb_ref[...]
    mu = jnp.mean(z, axis=1, keepdims=True)
    var = jnp.mean((z - mu) ** 2, axis=1, keepdims=True)
    zn = (z - mu) * lax.rsqrt(var + 1e-5) * cg_ref[...] + cbe_ref[...]
    h3 = jnp.maximum(zn, 0.0)
    y = jnp.dot(x_ref[...], w0_ref[...], preferred_element_type=jnp.float32)
    y += jnp.dot(h1_ref[...], w1_ref[...], preferred_element_type=jnp.float32)
    y += jnp.dot(h2_ref[...], w2_ref[...], preferred_element_type=jnp.float32)
    y += jnp.dot(h3, w3_ref[...], preferred_element_type=jnp.float32)
    y = y + bc1_ref[...]
    v = jnp.maximum(y * sg_ref[...] + bnb_ref[...], 0.0)
    u = jnp.sum(v * wc2_ref[...], axis=1)
    part = jnp.sum(u) * (1.0 / N)

    @pl.when(i == 0)
    def _():
        out_ref[...] = jnp.zeros_like(out_ref)

    g = i // (N // ROWS_BLK)
    row = lax.broadcasted_iota(jnp.int32, (2, D), 0)
    out_ref[...] += jnp.where(row == g, part, 0.0)


def _final(agg3, x, h1, h2, cw, cb, cg, cbe, wc1, bc1, bn_g, bn_b, wc2):
    h2d = 2 * D
    sg = (bn_g * (1.0 / jnp.sqrt(1.0 + 1e-5))).reshape(1, h2d)
    out = pl.pallas_call(
        _final_body,
        grid=(x.shape[0] // ROWS_BLK,),
        in_specs=[pl.BlockSpec((ROWS_BLK, D), lambda i: (i, 0))] * 4
        + [pl.BlockSpec((D, D), lambda i: (0, 0))]
        + [pl.BlockSpec((1, D), lambda i: (0, 0))] * 3
        + [pl.BlockSpec((D, h2d), lambda i: (0, 0))] * 4
        + [pl.BlockSpec((1, h2d), lambda i: (0, 0))] * 4,
        out_specs=pl.BlockSpec((2, D), lambda i: (0, 0)),
        out_shape=jax.ShapeDtypeStruct((2, D), jnp.float32),
    )(
        agg3, x, h1, h2,
        cw, cb.reshape(1, D), cg.reshape(1, D), cbe.reshape(1, D),
        wc1[0:D], wc1[D:2 * D], wc1[2 * D:3 * D], wc1[3 * D:4 * D],
        bc1.reshape(1, h2d), sg, bn_b.reshape(1, h2d), wc2.reshape(1, h2d),
    )
    return out[:, :1]


def kernel(graph_nodes, graph_edge_links, mask, conv0_W, conv0_b, conv0_g,
           conv0_be, conv1_W, conv1_b, conv1_g, conv1_be, conv2_W, conv2_b,
           conv2_g, conv2_be, Wc1, bc1, bn_g, bn_b, Wc2, bc2):
    x = graph_nodes.reshape(NT, D).astype(jnp.float32)
    ef = graph_edge_links.reshape(-1)
    convs = [(conv0_W, conv0_b, conv0_g, conv0_be),
             (conv1_W, conv1_b, conv1_g, conv1_be),
             (conv2_W, conv2_b, conv2_g, conv2_be)]
    pooled_halves = []
    for half in (0, 1):
        xh = x[half * 2 * N:(half + 1) * 2 * N]
        agg1 = _segsum(half, xh, ef)
        h1 = _gin_dense(agg1, *convs[0])
        agg2 = _segsum(half, h1, ef)
        h2 = _gin_dense(agg2, *convs[1])
        agg3 = _segsum(half, h2, ef)
        ph = _final(agg3, xh, h1, h2, *convs[2], Wc1, bc1, bn_g, bn_b, Wc2)
        pooled_halves.append(ph)
    return jnp.concatenate(pooled_halves, axis=0) + bc2


# speedup vs baseline: 1.0187x; 1.0187x over previous
import functools

import jax
import jax.numpy as jnp
from jax import lax
from jax.experimental import pallas as pl
from jax.experimental.pallas import tpu as pltpu
from jax.experimental.pallas import tpu_sc as plsc

B = 4
N = 10000
NT = B * N
D = 128
E = 160000

NC = 2
NS = 16
EPT = E // NS
W = 80
NWIN = EPT // W
NWT = 10
RPT = N // NWT
NBUF = 3


def _segsum_body(half, h_hbm, e_hbm, z_hbm, agg_hbm, src_f, dst_f, rows0,
                 rows1, rows2, acc_sh, g0, g1, g2, t0, t1, t2):
    c = lax.axis_index("c")
    s = lax.axis_index("s")
    rows = (rows0, rows1, rows2)
    gsem = (g0, g1, g2)
    ssem = (t0, t1, t2)
    if True:
        b = 2 * half + c

        off = b * 2 * E + s * EPT
        pltpu.async_copy(e_hbm.at[pl.ds(off, EPT)], src_f, g0)
        pltpu.async_copy(e_hbm.at[pl.ds(off + E, EPT)], dst_f, g1)

        @pl.when(s < NWT)
        def _zero_slice():
            pltpu.async_copy(z_hbm.at[pl.ds(s * RPT, RPT)],
                             acc_sh.at[pl.ds(s * RPT, RPT)], g2)

        pltpu.make_async_copy(e_hbm.at[pl.ds(off, EPT)], src_f, g0).wait()
        boff = c * N

        @pl.loop(0, EPT // 16)
        def _off(i):
            src_f[pl.ds(i * 16, 16)] = src_f[pl.ds(i * 16, 16)] + boff

        pltpu.make_async_copy(e_hbm.at[pl.ds(off + E, EPT)], dst_f, g1).wait()

        @pl.when(s < NWT)
        def _zero_wait():
            pltpu.make_async_copy(z_hbm.at[pl.ds(s * RPT, RPT)],
                                  acc_sh.at[pl.ds(s * RPT, RPT)], g2).wait()

        plsc.subcore_barrier()

        def _src(w):
            return src_f.at[pl.ds(w * W, W)]

        def _dst(w):
            return dst_f.at[pl.ds(w * W, W)]

        for j in range(NBUF):
            pltpu.async_copy(h_hbm.at[_src(j)], rows[j], gsem[j])

        @pl.loop(0, NWIN // NBUF)
        def _ring(k):
            w0 = NBUF * k
            for j in range(NBUF):
                pltpu.make_async_copy(
                    h_hbm.at[_src(w0 + j)], rows[j], gsem[j]).wait()
                pltpu.async_copy(
                    rows[j], acc_sh.at[_dst(w0 + j)], ssem[j], add=True)
            for j in range(NBUF):
                pltpu.make_async_copy(
                    rows[j], acc_sh.at[_dst(w0 + j)], ssem[j]).wait()

                @pl.when(w0 + j + NBUF < NWIN)
                def _prefetch():
                    pltpu.async_copy(
                        h_hbm.at[_src(w0 + j + NBUF)], rows[j], gsem[j])

        for j in range(NWIN % NBUF):
            w = (NWIN // NBUF) * NBUF + j
            pltpu.make_async_copy(h_hbm.at[_src(w)], rows[j], gsem[j]).wait()
            pltpu.sync_copy(rows[j], acc_sh.at[_dst(w)], add=True)

        plsc.subcore_barrier()

        @pl.when(s < NWT)
        def _writeback():
            pltpu.sync_copy(acc_sh.at[pl.ds(s * RPT, RPT)],
                            agg_hbm.at[pl.ds(c * N + s * RPT, RPT)])


@functools.cache
def _make_segsum(half):
    return pl.kernel(
        functools.partial(_segsum_body, half),
        out_type=jax.ShapeDtypeStruct((2 * N, D), jnp.float32),
        mesh=plsc.VectorSubcoreMesh(
            core_axis_name="c", subcore_axis_name="s",
            num_cores=NC, num_subcores=NS,
        ),
        scratch_types=[
            pltpu.VMEM((EPT,), jnp.int32),
            pltpu.VMEM((EPT,), jnp.int32),
            pltpu.VMEM((W, D), jnp.float32),
            pltpu.VMEM((W, D), jnp.float32),
            pltpu.VMEM((W, D), jnp.float32),
            pltpu.VMEM_SHARED((N, D), jnp.float32),
            pltpu.SemaphoreType.DMA,
            pltpu.SemaphoreType.DMA,
            pltpu.SemaphoreType.DMA,
            pltpu.SemaphoreType.DMA,
            pltpu.SemaphoreType.DMA,
            pltpu.SemaphoreType.DMA,
        ],
    )


def _segsum(half, h_half, ef):
    return _make_segsum(half)(h_half, ef, jnp.zeros((N, D), jnp.float32))


ROWS_BLK = 2000


def _gin_body(agg_ref, w_ref, b_ref, g_ref, be_ref, out_ref):
    z = jnp.dot(agg_ref[...], w_ref[...], preferred_element_type=jnp.float32)
    z = z + b_ref[...]
    mu = jnp.mean(z, axis=1, keepdims=True)
    var = jnp.mean((z - mu) ** 2, axis=1, keepdims=True)
    zn = (z - mu) * lax.rsqrt(var + 1e-5) * g_ref[...] + be_ref[...]
    out_ref[...] = jnp.maximum(zn, 0.0)


def _gin_dense(agg, w, b, g, be):
    nrows = agg.shape[0]
    return pl.pallas_call(
        _gin_body,
        grid=(nrows // ROWS_BLK,),
        in_specs=[
            pl.BlockSpec((ROWS_BLK, D), lambda i: (i, 0)),
            pl.BlockSpec((D, D), lambda i: (0, 0)),
            pl.BlockSpec((1, D), lambda i: (0, 0)),
            pl.BlockSpec((1, D), lambda i: (0, 0)),
            pl.BlockSpec((1, D), lambda i: (0, 0)),
        ],
        out_specs=pl.BlockSpec((ROWS_BLK, D), lambda i: (i, 0)),
        out_shape=jax.ShapeDtypeStruct((nrows, D), jnp.float32),
    )(agg, w, b.reshape(1, D), g.reshape(1, D), be.reshape(1, D))


def _final_body(agg_ref, x_ref, h1_ref, h2_ref, cw_ref, cb_ref, cg_ref,
                cbe_ref, w0_ref, w1_ref, w2_ref, w3_ref, bc1_ref, sg_ref,
                bnb_ref, wc2_ref, out_ref):
    i = pl.program_id(0)
    z = jnp.dot(agg_ref[...], cw_ref[...], preferred_element_type=jnp.float32)
    z = z + cb_ref[...]
    mu = jnp.mean(z, axis=1, keepdims=True)
    var = jnp.mean((z - mu) ** 2, axis=1, keepdims=True)
    zn = (z - mu) * lax.rsqrt(var + 1e-5) * cg_ref[...] + cbe_ref[...]
    h3 = jnp.maximum(zn, 0.0)
    y = jnp.dot(x_ref[...], w0_ref[...], preferred_element_type=jnp.float32)
    y += jnp.dot(h1_ref[...], w1_ref[...], preferred_element_type=jnp.float32)
    y += jnp.dot(h2_ref[...], w2_ref[...], preferred_element_type=jnp.float32)
    y += jnp.dot(h3, w3_ref[...], preferred_element_type=jnp.float32)
    y = y + bc1_ref[...]
    v = jnp.maximum(y * sg_ref[...] + bnb_ref[...], 0.0)
    u = jnp.sum(v * wc2_ref[...], axis=1)
    part = jnp.sum(u) * (1.0 / N)

    @pl.when(i == 0)
    def _():
        out_ref[...] = jnp.zeros_like(out_ref)

    g = i // (N // ROWS_BLK)
    row = lax.broadcasted_iota(jnp.int32, (2, D), 0)
    out_ref[...] += jnp.where(row == g, part, 0.0)


def _final(agg3, x, h1, h2, cw, cb, cg, cbe, wc1, bc1, bn_g, bn_b, wc2):
    h2d = 2 * D
    sg = (bn_g * (1.0 / jnp.sqrt(1.0 + 1e-5))).reshape(1, h2d)
    out = pl.pallas_call(
        _final_body,
        grid=(x.shape[0] // ROWS_BLK,),
        in_specs=[pl.BlockSpec((ROWS_BLK, D), lambda i: (i, 0))] * 4
        + [pl.BlockSpec((D, D), lambda i: (0, 0))]
        + [pl.BlockSpec((1, D), lambda i: (0, 0))] * 3
        + [pl.BlockSpec((D, h2d), lambda i: (0, 0))] * 4
        + [pl.BlockSpec((1, h2d), lambda i: (0, 0))] * 4,
        out_specs=pl.BlockSpec((2, D), lambda i: (0, 0)),
        out_shape=jax.ShapeDtypeStruct((2, D), jnp.float32),
    )(
        agg3, x, h1, h2,
        cw, cb.reshape(1, D), cg.reshape(1, D), cbe.reshape(1, D),
        wc1[0:D], wc1[D:2 * D], wc1[2 * D:3 * D], wc1[3 * D:4 * D],
        bc1.reshape(1, h2d), sg, bn_b.reshape(1, h2d), wc2.reshape(1, h2d),
    )
    return out[:, :1]


def kernel(graph_nodes, graph_edge_links, mask, conv0_W, conv0_b, conv0_g,
           conv0_be, conv1_W, conv1_b, conv1_g, conv1_be, conv2_W, conv2_b,
           conv2_g, conv2_be, Wc1, bc1, bn_g, bn_b, Wc2, bc2):
    x = graph_nodes.reshape(NT, D).astype(jnp.float32)
    ef = graph_edge_links.reshape(-1)
    convs = [(conv0_W, conv0_b, conv0_g, conv0_be),
             (conv1_W, conv1_b, conv1_g, conv1_be),
             (conv2_W, conv2_b, conv2_g, conv2_be)]
    pooled_halves = []
    for half in (0, 1):
        xh = x[half * 2 * N:(half + 1) * 2 * N]
        agg1 = _segsum(half, xh, ef)
        h1 = _gin_dense(agg1, *convs[0])
        agg2 = _segsum(half, h1, ef)
        h2 = _gin_dense(agg2, *convs[1])
        agg3 = _segsum(half, h2, ef)
        ph = _final(agg3, xh, h1, h2, *convs[2], Wc1, bc1, bn_g, bn_b, Wc2)
        pooled_halves.append(ph)
    return jnp.concatenate(pooled_halves, axis=0) + bc2
